# baseline (speedup 1.0000x reference)
import functools

import jax
import jax.numpy as jnp
from jax import lax
from jax.experimental import pallas as pl
from jax.experimental.pallas import tpu as pltpu
from jax.experimental.pallas import tpu_sc as plsc

L = 16
NB = 64
HW_ROWS = 512
HW_COLS = 512
CHUNK_ROWS = 64
N_CHUNKS = HW_ROWS // CHUNK_ROWS
WPC = CHUNK_ROWS // 4
IMGS_PER_TILE = 2


def _sc_body(value_hbm, ptab_hbm, out_hbm,
             vbuf0, vbuf1, o_r0, o_g0, o_b0, o_r1, o_g1, o_b1, tab, red,
             si0, si1, so0, so1):
    nc = 2
    wid = lax.axis_index("s") * nc + lax.axis_index("c")
    pltpu.sync_copy(ptab_hbm, tab)
    iota = lax.iota(jnp.int32, L)
    outw = out_hbm.bitcast(jnp.int32)

    def _all_reduce(vec, op):
        for k in (1, 2, 4, 8):
            red[pl.ds(0, L)] = vec
            partner = plsc.load_gather(red, [jnp.bitwise_xor(iota, k)])
            vec = op(vec, partner)
        return vec

    def _scan_buf(buf, mn, mx):
        @plsc.parallel_loop(0, CHUNK_ROWS, unroll=4,
                            carry=(mn, mn, mn, mn, mx, mx, mx, mx))
        def accs(r, acc):
            m0, m1, m2, m3, x0, x1, x2, x3 = acc
            ms, xs = [m0, m1, m2, m3], [x0, x1, x2, x3]
            for q in range(HW_COLS // L):
                v = buf[r, pl.ds(q * L, L)]
                ms[q % 4] = jnp.minimum(ms[q % 4], v)
                xs[q % 4] = jnp.maximum(xs[q % 4], v)
            return (*ms, *xs)

        m0, m1, m2, m3, x0, x1, x2, x3 = accs
        mn = jnp.minimum(jnp.minimum(m0, m1), jnp.minimum(m2, m3))
        mx = jnp.maximum(jnp.maximum(x0, x1), jnp.maximum(x2, x3))
        return mn, mx

    def _compute_chunk(buf, st_r, st_g, st_b, sb, nb):
        @plsc.parallel_loop(0, WPC * (HW_COLS // L), unroll=8)
        def _(t):
            r4 = t >> 5
            col0 = (t & 31) << 4
            cs = []
            for s in range(4):
                v = buf[r4 * 4 + s, pl.ds(col0, L)]
                idx = (v * sb + nb).astype(jnp.int32)
                cs.append(plsc.load_gather(tab, [idx]))
            c0, c1, c2, c3 = cs
            accr = ((c0 & 0xFF)
                    | ((c1 & 0xFF) << 8)
                    | ((c2 & 0xFF) << 16)
                    | (c3 << 24))
            accg = (((c0 >> 8) & 0xFF)
                    | (c1 & 0xFF00)
                    | ((c2 & 0xFF00) << 8)
                    | ((c3 & 0xFF00) << 16))
            accb = (((c0 >> 16) & 0xFF)
                    | ((c1 >> 8) & 0xFF00)
                    | (c2 & 0xFF0000)
                    | ((c3 & 0xFF0000) << 8))
            st_r[r4, pl.ds(col0, L)] = accr
            st_g[r4, pl.ds(col0, L)] = accg
            st_b[r4, pl.ds(col0, L)] = accb

    for img in range(IMGS_PER_TILE):
        i = wid * IMGS_PER_TILE + img

        def src(c):
            return value_hbm.at[i, 0, pl.ds(c * CHUNK_ROWS, CHUNK_ROWS)]

        def dsts(c):
            w0 = c * WPC
            return (outw.at[i, pl.ds(w0, WPC)],
                    outw.at[64 + i, pl.ds(w0, WPC)],
                    outw.at[128 + i, pl.ds(w0, WPC)])

        pltpu.async_copy(src(0), vbuf0, si0)

        def pass_a(k, carry):
            mn, mx = carry
            c0 = 2 * k
            pltpu.make_async_copy(src(c0), vbuf0, si0).wait()
            pltpu.async_copy(src(c0 + 1), vbuf1, si1)
            mn, mx = _scan_buf(vbuf0, mn, mx)
            pltpu.make_async_copy(src(c0 + 1), vbuf1, si1).wait()

            @pl.when(k < N_CHUNKS // 2 - 1)
            def _():
                pltpu.async_copy(src(c0 + 2), vbuf0, si0)

            return _scan_buf(vbuf1, mn, mx)

        big = jnp.full((L,), jnp.inf, jnp.float32)
        mn, mx = lax.fori_loop(0, N_CHUNKS // 2, pass_a, (big, -big))
        mnb = _all_reduce(mn, jnp.minimum)
        mxb = _all_reduce(mx, jnp.maximum)
        sb = 255.0 / jnp.maximum(mxb - mnb, 1e-5)
        nb = -(mnb * sb)

        pltpu.async_copy(src(0), vbuf0, si0)

        def pass_b(k, z):
            for half in range(2):
                c = 2 * k + half
                buf, sin = (vbuf0, si0) if half == 0 else (vbuf1, si1)
                nbuf, nsin = (vbuf1, si1) if half == 0 else (vbuf0, si0)
                stg = (o_r0, o_g0, o_b0) if half == 0 else (o_r1, o_g1, o_b1)
                sout = so0 if half == 0 else so1
                pltpu.make_async_copy(src(c), buf, sin).wait()

                @pl.when(c < N_CHUNKS - 1)
                def _():
                    pltpu.async_copy(src(c + 1), nbuf, nsin)

                @pl.when(k > 0)
                def _():
                    for sref, dref in zip(stg, dsts(c)):
                        pltpu.make_async_copy(sref, dref, sout).wait()

                _compute_chunk(buf, *stg, sb, nb)
                for sref, dref in zip(stg, dsts(c)):
                    pltpu.async_copy(sref, dref, sout)
            return z

        lax.fori_loop(0, N_CHUNKS // 2, pass_b, 0)
        for half in range(2):
            c = N_CHUNKS - 2 + half
            stg = (o_r0, o_g0, o_b0) if half == 0 else (o_r1, o_g1, o_b1)
            sout = so0 if half == 0 else so1
            for sref, dref in zip(stg, dsts(c)):
                pltpu.make_async_copy(sref, dref, sout).wait()


@jax.jit
def _colormap_sc(value, ptab):
    f = pl.kernel(
        _sc_body,
        out_type=jax.ShapeDtypeStruct((3 * NB, HW_ROWS, HW_COLS), jnp.uint8),
        mesh=plsc.VectorSubcoreMesh(core_axis_name="c", subcore_axis_name="s"),
        compiler_params=pltpu.CompilerParams(needs_layout_passes=False),
        scratch_types=[
            pltpu.VMEM((CHUNK_ROWS, HW_COLS), jnp.float32),
            pltpu.VMEM((CHUNK_ROWS, HW_COLS), jnp.float32),
            pltpu.VMEM((WPC, HW_COLS), jnp.int32),
            pltpu.VMEM((WPC, HW_COLS), jnp.int32),
            pltpu.VMEM((WPC, HW_COLS), jnp.int32),
            pltpu.VMEM((WPC, HW_COLS), jnp.int32),
            pltpu.VMEM((WPC, HW_COLS), jnp.int32),
            pltpu.VMEM((WPC, HW_COLS), jnp.int32),
            pltpu.VMEM((256,), jnp.int32),
            pltpu.VMEM((L,), jnp.float32),
            pltpu.SemaphoreType.DMA,
            pltpu.SemaphoreType.DMA,
            pltpu.SemaphoreType.DMA,
            pltpu.SemaphoreType.DMA,
        ],
    )
    return f(value, ptab).reshape(NB, 3, HW_ROWS, HW_COLS)


def kernel(value, cmap):
    c32 = cmap.astype(jnp.int32)
    ptab = c32[:, 0] | (c32[:, 1] << 8) | (c32[:, 2] << 16)
    return _colormap_sc(value, ptab)


# speedup vs baseline: 1.0594x; 1.0594x over previous
import functools

import jax
import jax.numpy as jnp
from jax import lax
from jax.experimental import pallas as pl
from jax.experimental.pallas import tpu as pltpu
from jax.experimental.pallas import tpu_sc as plsc

L = 16
NB = 64
HW_ROWS = 512
HW_COLS = 512
CHUNK_ROWS = 64
N_CHUNKS = HW_ROWS // CHUNK_ROWS
WPC = CHUNK_ROWS // 4
IMGS_PER_TILE = 2


def _sc_body(value_hbm, ptab_hbm, out_hbm,
             vbuf0, vbuf1, o_r0, o_g0, o_b0, o_r1, o_g1, o_b1,
             t_rg_lo, t_rg_hi, t_b_lo, t_b_hi, red,
             si0, si1, so0, so1):
    nc = 2
    wid = lax.axis_index("s") * nc + lax.axis_index("c")
    pltpu.sync_copy(ptab_hbm.at[0], t_rg_lo)
    pltpu.sync_copy(ptab_hbm.at[1], t_rg_hi)
    pltpu.sync_copy(ptab_hbm.at[2], t_b_lo)
    pltpu.sync_copy(ptab_hbm.at[3], t_b_hi)
    iota = lax.iota(jnp.int32, L)
    outw = out_hbm.bitcast(jnp.int32)

    def _all_reduce(vec, op):
        for k in (1, 2, 4, 8):
            red[pl.ds(0, L)] = vec
            partner = plsc.load_gather(red, [jnp.bitwise_xor(iota, k)])
            vec = op(vec, partner)
        return vec

    def _scan_buf(buf, mn, mx):
        @plsc.parallel_loop(0, CHUNK_ROWS, unroll=2,
                            carry=(mn, mn, mn, mn, mx, mx, mx, mx))
        def accs(r, acc):
            m0, m1, m2, m3, x0, x1, x2, x3 = acc
            ms, xs = [m0, m1, m2, m3], [x0, x1, x2, x3]
            for q in range(HW_COLS // L):
                v = buf[r, pl.ds(q * L, L)]
                ms[q % 4] = jnp.minimum(ms[q % 4], v)
                xs[q % 4] = jnp.maximum(xs[q % 4], v)
            return (*ms, *xs)

        m0, m1, m2, m3, x0, x1, x2, x3 = accs
        mn = jnp.minimum(jnp.minimum(m0, m1), jnp.minimum(m2, m3))
        mx = jnp.maximum(jnp.maximum(x0, x1), jnp.maximum(x2, x3))
        return mn, mx

    def _compute_chunk(buf, st_r, st_g, st_b, sb, nb):
        @plsc.parallel_loop(0, WPC * (HW_COLS // L), unroll=4)
        def _(t):
            r4 = t >> 5
            col0 = (t & 31) << 4
            idxs = []
            for s in range(4):
                v = buf[r4 * 4 + s, pl.ds(col0, L)]
                idxs.append((v * sb + nb).astype(jnp.int32))
            p01 = (plsc.load_gather(t_rg_lo, [idxs[0]])
                   | plsc.load_gather(t_rg_hi, [idxs[1]]))
            p23 = (plsc.load_gather(t_rg_lo, [idxs[2]])
                   | plsc.load_gather(t_rg_hi, [idxs[3]]))
            b01 = (plsc.load_gather(t_b_lo, [idxs[0]])
                   | plsc.load_gather(t_b_hi, [idxs[1]]))
            b23 = (plsc.load_gather(t_b_lo, [idxs[2]])
                   | plsc.load_gather(t_b_hi, [idxs[3]]))
            st_r[r4, pl.ds(col0, L)] = (p01 & 0xFFFF) | (p23 << 16)
            st_g[r4, pl.ds(col0, L)] = (
                lax.shift_right_logical(p01, 16)
                | (p23 & jnp.int32(-65536)))
            st_b[r4, pl.ds(col0, L)] = b01 | (b23 << 16)

    for img in range(IMGS_PER_TILE):
        i = wid * IMGS_PER_TILE + img

        def src(c):
            return value_hbm.at[i, 0, pl.ds(c * CHUNK_ROWS, CHUNK_ROWS)]

        def dsts(c):
            w0 = c * WPC
            return (outw.at[i, pl.ds(w0, WPC)],
                    outw.at[64 + i, pl.ds(w0, WPC)],
                    outw.at[128 + i, pl.ds(w0, WPC)])

        pltpu.async_copy(src(0), vbuf0, si0)

        def pass_a(k, carry):
            mn, mx = carry
            c0 = 2 * k
            pltpu.make_async_copy(src(c0), vbuf0, si0).wait()
            pltpu.async_copy(src(c0 + 1), vbuf1, si1)
            mn, mx = _scan_buf(vbuf0, mn, mx)
            pltpu.make_async_copy(src(c0 + 1), vbuf1, si1).wait()

            @pl.when(k < N_CHUNKS // 2 - 1)
            def _():
                pltpu.async_copy(src(c0 + 2), vbuf0, si0)

            return _scan_buf(vbuf1, mn, mx)

        big = jnp.full((L,), jnp.inf, jnp.float32)
        mn, mx = lax.fori_loop(0, N_CHUNKS // 2, pass_a, (big, -big))
        mnb = _all_reduce(mn, jnp.minimum)
        mxb = _all_reduce(mx, jnp.maximum)
        sb = 255.0 / jnp.maximum(mxb - mnb, 1e-5)
        nb = -(mnb * sb)

        pltpu.async_copy(src(0), vbuf0, si0)

        def pass_b(k, z):
            for half in range(2):
                c = 2 * k + half
                buf, sin = (vbuf0, si0) if half == 0 else (vbuf1, si1)
                nbuf, nsin = (vbuf1, si1) if half == 0 else (vbuf0, si0)
                stg = (o_r0, o_g0, o_b0) if half == 0 else (o_r1, o_g1, o_b1)
                sout = so0 if half == 0 else so1
                pltpu.make_async_copy(src(c), buf, sin).wait()

                @pl.when(c < N_CHUNKS - 1)
                def _():
                    pltpu.async_copy(src(c + 1), nbuf, nsin)

                @pl.when(k > 0)
                def _():
                    for sref, dref in zip(stg, dsts(c)):
                        pltpu.make_async_copy(sref, dref, sout).wait()

                _compute_chunk(buf, *stg, sb, nb)
                for sref, dref in zip(stg, dsts(c)):
                    pltpu.async_copy(sref, dref, sout)
            return z

        lax.fori_loop(0, N_CHUNKS // 2, pass_b, 0)
        for half in range(2):
            c = N_CHUNKS - 2 + half
            stg = (o_r0, o_g0, o_b0) if half == 0 else (o_r1, o_g1, o_b1)
            sout = so0 if half == 0 else so1
            for sref, dref in zip(stg, dsts(c)):
                pltpu.make_async_copy(sref, dref, sout).wait()


@jax.jit
def _colormap_sc(value, ptab):
    f = pl.kernel(
        _sc_body,
        out_type=jax.ShapeDtypeStruct((3 * NB, HW_ROWS, HW_COLS), jnp.uint8),
        mesh=plsc.VectorSubcoreMesh(core_axis_name="c", subcore_axis_name="s"),
        compiler_params=pltpu.CompilerParams(needs_layout_passes=False),
        scratch_types=[
            pltpu.VMEM((CHUNK_ROWS, HW_COLS), jnp.float32),
            pltpu.VMEM((CHUNK_ROWS, HW_COLS), jnp.float32),
            pltpu.VMEM((WPC, HW_COLS), jnp.int32),
            pltpu.VMEM((WPC, HW_COLS), jnp.int32),
            pltpu.VMEM((WPC, HW_COLS), jnp.int32),
            pltpu.VMEM((WPC, HW_COLS), jnp.int32),
            pltpu.VMEM((WPC, HW_COLS), jnp.int32),
            pltpu.VMEM((WPC, HW_COLS), jnp.int32),
            pltpu.VMEM((256,), jnp.int32),
            pltpu.VMEM((256,), jnp.int32),
            pltpu.VMEM((256,), jnp.int32),
            pltpu.VMEM((256,), jnp.int32),
            pltpu.VMEM((L,), jnp.float32),
            pltpu.SemaphoreType.DMA,
            pltpu.SemaphoreType.DMA,
            pltpu.SemaphoreType.DMA,
            pltpu.SemaphoreType.DMA,
        ],
    )
    return f(value, ptab).reshape(NB, 3, HW_ROWS, HW_COLS)


def kernel(value, cmap):
    c32 = cmap.astype(jnp.int32)
    r, g, b = c32[:, 0], c32[:, 1], c32[:, 2]
    ptab = jnp.stack([
        r | (g << 16),
        (r << 8) | (g << 24),
        b,
        b << 8,
    ])
    return _colormap_sc(value, ptab)
